# baseline (speedup 1.0000x reference)
import functools

import jax
import jax.numpy as jnp
from jax import lax
from jax.experimental import pallas as pl
from jax.experimental.pallas import tpu as pltpu
from jax.experimental.pallas import tpu_sc as plsc

BATCH = 4096
HIST = 200
DIM = 32
VOCAB = 1000000
NC, NS = 2, 16
NW = NC * NS
PACK = 128 // DIM
WRM_ROWS = VOCAB // PACK
NBLK = VOCAB // 128
TAIL = VOCAB - NBLK * 128
BLK_PER_W = NBLK // NW
BLK_REM = NBLK - BLK_PER_W * NW
HH = HIST // 8
BB = BATCH // 128
TILES_PER_W = HH * BB // NW

_MESH = plsc.VectorSubcoreMesh(core_axis_name="c", subcore_axis_name="s")


def _iota16():
    return lax.iota(jnp.int32, 16)


@functools.partial(
    pl.kernel,
    mesh=_MESH,
    out_type=jax.ShapeDtypeStruct((WRM_ROWS, 128), jnp.float32),
    scratch_types=[
        pltpu.VMEM((DIM, 128), jnp.float32),
        pltpu.VMEM((DIM, 128), jnp.float32),
    ],
    compiler_params=pltpu.CompilerParams(needs_layout_passes=False),
)
def _transpose_table(w_t, wt_tail, wrm, src, dst):
    wid = lax.axis_index("s") * NC + lax.axis_index("c")
    nblk = jnp.where(wid < BLK_REM, BLK_PER_W + 1, BLK_PER_W)

    def do_block(c, nrow):
        pltpu.sync_copy(w_t.at[:, pl.ds(pl.multiple_of(c * 128, 128), 128)], src)
        def do_row(r, _):
            for g in range(8):
                rows = _iota16() + (g % 2) * 16
                cols = jnp.full((16,), r * PACK + g // 2, jnp.int32)
                dst[r, pl.ds(g * 16, 16)] = plsc.load_gather(src, [rows, cols])
            return 0

        lax.fori_loop(0, nrow, do_row, 0, unroll=2)
        pltpu.sync_copy(dst.at[pl.ds(0, nrow), :], wrm.at[pl.ds(pl.multiple_of(c * DIM, DIM), nrow), :])

    def body(t, _):
        do_block(t * NW + wid, DIM)
        return 0

    lax.fori_loop(0, nblk, body, 0)

    @pl.when(wid == NW - 1)
    def _():
        nrow = TAIL // PACK
        pltpu.sync_copy(wt_tail, dst.at[pl.ds(0, nrow), :])
        pltpu.sync_copy(
            dst.at[pl.ds(0, nrow), :], wrm.at[pl.ds(WRM_ROWS - nrow, nrow), :]
        )


@functools.partial(
    pl.kernel,
    mesh=_MESH,
    out_type=jax.ShapeDtypeStruct((HIST, DIM, BATCH), jnp.float32),
    scratch_types=[
        pltpu.VMEM((8, 128), jnp.int32),
        pltpu.VMEM((128,), jnp.int32),
        pltpu.VMEM((128, 128), jnp.float32),
        pltpu.VMEM((DIM, 128), jnp.float32),
        pltpu.SemaphoreType.DMA,
    ],
    compiler_params=pltpu.CompilerParams(needs_layout_passes=False),
)
def _gather(tok_t, wrm, out, tokv, giv, gbuf, obuf, sem):
    wid = lax.axis_index("s") * NC + lax.axis_index("c")

    def tile_body(t, _):
        tile = t * NW + wid
        hh = tile // BB
        bb = tile - hh * BB
        pltpu.sync_copy(tok_t.at[pl.ds(pl.multiple_of(hh * 8, 8), 8), pl.ds(pl.multiple_of(bb * 128, 128), 128)], tokv)

        def h_body(hs, _):
            for g in range(8):
                giv[pl.ds(g * 16, 16)] = lax.shift_right_logical(
                    tokv[hs, pl.ds(g * 16, 16)], 2
                )
            pltpu.async_copy(wrm.at[giv], gbuf, sem).wait()
            def d_body(d, _):
                for g in range(8):
                    rows = _iota16() + g * 16
                    sub = lax.bitwise_and(tokv[hs, pl.ds(g * 16, 16)], 3)
                    cols = sub * DIM + d
                    obuf[d, pl.ds(g * 16, 16)] = plsc.load_gather(
                        gbuf, [rows, cols]
                    )
                return 0

            lax.fori_loop(0, DIM, d_body, 0, unroll=2)
            pltpu.sync_copy(
                obuf, out.at[hh * 8 + hs, :, pl.ds(pl.multiple_of(bb * 128, 128), 128)]
            )
            return 0

        lax.fori_loop(0, 8, h_body, 0)
        return 0

    lax.fori_loop(0, TILES_PER_W, tile_body, 0)


def kernel(token_ids, weight):
    wt_tail = weight[NBLK * 128 :].reshape(TAIL // PACK, 128)
    wrm = _transpose_table(weight.T, wt_tail)
    out_t = _gather(token_ids.T, wrm)
    return out_t.transpose(2, 0, 1)


# speedup vs baseline: 1.5165x; 1.5165x over previous
import functools

import jax
import jax.numpy as jnp
from jax import lax
from jax.experimental import pallas as pl
from jax.experimental.pallas import tpu as pltpu
from jax.experimental.pallas import tpu_sc as plsc

BATCH = 4096
HIST = 200
DIM = 32
VOCAB = 1000000
NC, NS = 2, 16
NW = NC * NS
PACK = 128 // DIM
WRM_ROWS = VOCAB // PACK
NBLK = VOCAB // 128
TAIL = VOCAB - NBLK * 128
BLK_PER_W = NBLK // NW
BLK_REM = NBLK - BLK_PER_W * NW
BB = BATCH // 128

_MESH = plsc.VectorSubcoreMesh(core_axis_name="c", subcore_axis_name="s")


def _iota16():
    return lax.iota(jnp.int32, 16)


def _transpose_block(src, dst, nrow):
    for r in range(nrow):
        for g in range(8):
            rows = _iota16() + (g % 2) * 16
            cols = jnp.full((16,), r * PACK + g // 2, jnp.int32)
            dst[r, pl.ds(g * 16, 16)] = plsc.load_gather(src, [rows, cols])


@functools.partial(
    pl.kernel,
    mesh=_MESH,
    out_type=jax.ShapeDtypeStruct((WRM_ROWS, 128), jnp.float32),
    scratch_types=[
        pltpu.VMEM((DIM, 128), jnp.float32),
        pltpu.VMEM((DIM, 128), jnp.float32),
        pltpu.VMEM((DIM, 128), jnp.float32),
        pltpu.VMEM((DIM, 128), jnp.float32),
        pltpu.SemaphoreType.DMA,
        pltpu.SemaphoreType.DMA,
        pltpu.SemaphoreType.DMA,
        pltpu.SemaphoreType.DMA,
    ],
    compiler_params=pltpu.CompilerParams(needs_layout_passes=False),
)
def _transpose_table(w_t, wt_tail, wrm, s0, s1, d0, d1, si0, si1, so0, so1):
    wid = lax.axis_index("s") * NC + lax.axis_index("c")

    def in_slice(c):
        return w_t.at[:, pl.ds(pl.multiple_of(c * 128, 128), 128)]

    def out_slice(c):
        return wrm.at[pl.ds(pl.multiple_of(c * DIM, DIM), DIM), :]

    def blk(t):
        return t * NW + wid

    pltpu.async_copy(in_slice(blk(0)), s0, si0)
    pltpu.async_copy(in_slice(blk(1)), s1, si1)

    def pair_body(p, _):
        t0 = 2 * p

        def stage(t, s, d, si, so):
            pltpu.make_async_copy(in_slice(blk(t)), s, si).wait()

            @pl.when(p > 0)
            def _():
                pltpu.make_async_copy(d, out_slice(blk(t - 2)), so).wait()

            _transpose_block(s, d, DIM)
            pltpu.async_copy(d, out_slice(blk(t)), so)

            @pl.when(t + 2 < BLK_PER_W)
            def _():
                pltpu.async_copy(in_slice(blk(t + 2)), s, si)

        stage(t0, s0, d0, si0, so0)
        stage(t0 + 1, s1, d1, si1, so1)
        return 0

    lax.fori_loop(0, BLK_PER_W // 2, pair_body, 0)
    pltpu.make_async_copy(d0, out_slice(blk(BLK_PER_W - 2)), so0).wait()
    pltpu.make_async_copy(d1, out_slice(blk(BLK_PER_W - 1)), so1).wait()

    @pl.when(wid < BLK_REM)
    def _():
        c = BLK_PER_W * NW + wid
        pltpu.sync_copy(in_slice(c), s0)
        _transpose_block(s0, d0, DIM)
        pltpu.sync_copy(d0, out_slice(c))

    @pl.when(wid == NW - 1)
    def _():
        nrow = TAIL // PACK
        pltpu.sync_copy(wt_tail, s1.at[pl.ds(0, nrow), :])
        pltpu.sync_copy(
            s1.at[pl.ds(0, nrow), :], wrm.at[pl.ds(WRM_ROWS - nrow, nrow), :]
        )


@functools.partial(
    pl.kernel,
    mesh=_MESH,
    out_type=jax.ShapeDtypeStruct((HIST, PACK, BB, 8, 128), jnp.float32),
    scratch_types=[
        pltpu.VMEM((128,), jnp.int32),
        pltpu.VMEM((128,), jnp.int32),
        pltpu.VMEM((128, DIM), jnp.float32),
        pltpu.VMEM((128, DIM), jnp.float32),
        pltpu.VMEM((PACK, 8, 128), jnp.float32),
        pltpu.VMEM((PACK, 8, 128), jnp.float32),
        pltpu.SemaphoreType.DMA,
        pltpu.SemaphoreType.DMA,
        pltpu.SemaphoreType.DMA,
        pltpu.SemaphoreType.DMA,
        pltpu.SemaphoreType.DMA,
        pltpu.SemaphoreType.DMA,
    ],
    compiler_params=pltpu.CompilerParams(
        use_tc_tiling_on_sc=False, needs_layout_passes=False
    ),
)
def _gather(
    tok, wlin, out, i0, i1, g0, g1, o0, o1, sI0, sI1, sG0, sG1, sO0, sO1
):
    wid = lax.axis_index("s") * NC + lax.axis_index("c")

    def tok_slice(h):
        return tok.at[pl.ds(pl.multiple_of(h * BATCH + wid * 128, 128), 128)]

    def out_slice(h):
        return out.at[h, :, wid]

    def extract(g, o):
        for d in range(DIM):
            for q in range(8):
                rows = _iota16() + q * 16
                cols = jnp.full((16,), d, jnp.int32)
                o[d // 8, d % 8, pl.ds(q * 16, 16)] = plsc.load_gather(
                    g, [rows, cols]
                )

    pltpu.async_copy(tok_slice(0), i0, sI0)
    pltpu.async_copy(tok_slice(1), i1, sI1)
    pltpu.make_async_copy(tok_slice(0), i0, sI0).wait()
    pltpu.async_copy(wlin.at[i0], g0, sG0)

    def stage(p, h, iv, gb, ob, sI, sG, sO):
        pltpu.make_async_copy(wlin.at[iv], gb, sG).wait()

        @pl.when(h + 2 < HIST)
        def _():
            pltpu.async_copy(tok_slice(h + 2), iv, sI)

        @pl.when(p > 0)
        def _():
            pltpu.make_async_copy(ob, out_slice(h - 2), sO).wait()

        extract(gb, ob)
        pltpu.async_copy(ob, out_slice(h), sO)

    def pair_body(p, _):
        h0 = 2 * p
        pltpu.make_async_copy(tok_slice(h0 + 1), i1, sI1).wait()
        pltpu.async_copy(wlin.at[i1], g1, sG1)
        stage(p, h0, i0, g0, o0, sI0, sG0, sO0)
        @pl.when(h0 + 2 < HIST)
        def _():
            pltpu.make_async_copy(tok_slice(h0 + 2), i0, sI0).wait()
            pltpu.async_copy(wlin.at[i0], g0, sG0)

        stage(p, h0 + 1, i1, g1, o1, sI1, sG1, sO1)
        return 0

    lax.fori_loop(0, HIST // 2, pair_body, 0)
    pltpu.make_async_copy(o0, out_slice(HIST - 2), sO0).wait()
    pltpu.make_async_copy(o1, out_slice(HIST - 1), sO1).wait()


def kernel(token_ids, weight):
    wt_tail = weight[NBLK * 128 :].reshape(TAIL // PACK, 128)
    wrm = _transpose_table(weight.T, wt_tail)
    wlin = wrm.reshape(VOCAB, DIM)
    tok = token_ids.T.reshape(BATCH * HIST)
    out5 = _gather(tok, wlin)
    return out5.transpose(2, 4, 0, 1, 3).reshape(BATCH, HIST, DIM)


# speedup vs baseline: 2.8185x; 1.8585x over previous
import functools

import jax
import jax.numpy as jnp
from jax import lax
from jax.experimental import pallas as pl
from jax.experimental.pallas import tpu as pltpu
from jax.experimental.pallas import tpu_sc as plsc

BATCH = 4096
HIST = 200
DIM = 32
VOCAB = 1000000
NC, NS = 2, 16
NW = NC * NS
PACK = 128 // DIM
WRM_ROWS = VOCAB // PACK
NBLK = VOCAB // 128
TAIL = VOCAB - NBLK * 128
BLK_PER_W = NBLK // NW
BLK_REM = NBLK - BLK_PER_W * NW
BB = BATCH // 128

_MESH = plsc.VectorSubcoreMesh(core_axis_name="c", subcore_axis_name="s")


def _iota16():
    return lax.iota(jnp.int32, 16)


def _transpose_block(src, dst):
    for l0 in range(0, 128, 16):
        lvec = _iota16() + l0
        rquart = lax.shift_right_logical(lvec, 2)
        lmod = lax.bitwise_and(lvec, 3) * DIM
        for d0 in range(DIM):
            dvec = lax.bitwise_and(_iota16() + d0, DIM - 1)
            vals = plsc.load_gather(src, [dvec, lvec])
            plsc.store_scatter(dst, [rquart, lmod + dvec], vals)


@functools.partial(
    pl.kernel,
    mesh=_MESH,
    out_type=jax.ShapeDtypeStruct((WRM_ROWS, 128), jnp.float32),
    scratch_types=[
        pltpu.VMEM((DIM, 128), jnp.float32),
        pltpu.VMEM((DIM, 128), jnp.float32),
        pltpu.VMEM((DIM, 128), jnp.float32),
        pltpu.VMEM((DIM, 128), jnp.float32),
        pltpu.SemaphoreType.DMA,
        pltpu.SemaphoreType.DMA,
        pltpu.SemaphoreType.DMA,
        pltpu.SemaphoreType.DMA,
    ],
    compiler_params=pltpu.CompilerParams(needs_layout_passes=False),
)
def _transpose_table(w_t, wt_tail, wrm, s0, s1, d0, d1, si0, si1, so0, so1):
    wid = lax.axis_index("s") * NC + lax.axis_index("c")

    def in_slice(c):
        return w_t.at[:, pl.ds(pl.multiple_of(c * 128, 128), 128)]

    def out_slice(c):
        return wrm.at[pl.ds(pl.multiple_of(c * DIM, DIM), DIM), :]

    def blk(t):
        return t * NW + wid

    pltpu.async_copy(in_slice(blk(0)), s0, si0)
    pltpu.async_copy(in_slice(blk(1)), s1, si1)

    def pair_body(p, _):
        t0 = 2 * p

        def stage(t, s, d, si, so):
            pltpu.make_async_copy(in_slice(blk(t)), s, si).wait()

            @pl.when(p > 0)
            def _():
                pltpu.make_async_copy(d, out_slice(blk(t - 2)), so).wait()

            _transpose_block(s, d)
            pltpu.async_copy(d, out_slice(blk(t)), so)

            @pl.when(t + 2 < BLK_PER_W)
            def _():
                pltpu.async_copy(in_slice(blk(t + 2)), s, si)

        stage(t0, s0, d0, si0, so0)
        stage(t0 + 1, s1, d1, si1, so1)
        return 0

    lax.fori_loop(0, BLK_PER_W // 2, pair_body, 0)
    pltpu.make_async_copy(d0, out_slice(blk(BLK_PER_W - 2)), so0).wait()
    pltpu.make_async_copy(d1, out_slice(blk(BLK_PER_W - 1)), so1).wait()

    @pl.when(wid < BLK_REM)
    def _():
        c = BLK_PER_W * NW + wid
        pltpu.sync_copy(in_slice(c), s0)
        _transpose_block(s0, d0)
        pltpu.sync_copy(d0, out_slice(c))

    @pl.when(wid == NW - 1)
    def _():
        nrow = TAIL // PACK
        pltpu.sync_copy(wt_tail, s1.at[pl.ds(0, nrow), :])
        pltpu.sync_copy(
            s1.at[pl.ds(0, nrow), :], wrm.at[pl.ds(WRM_ROWS - nrow, nrow), :]
        )


@functools.partial(
    pl.kernel,
    mesh=_MESH,
    out_type=jax.ShapeDtypeStruct((HIST, PACK, BB, 8, 128), jnp.float32),
    scratch_types=[
        pltpu.VMEM((128,), jnp.int32),
        pltpu.VMEM((128,), jnp.int32),
        pltpu.VMEM((128, DIM), jnp.float32),
        pltpu.VMEM((128, DIM), jnp.float32),
        pltpu.VMEM((PACK, 8, 128), jnp.float32),
        pltpu.VMEM((PACK, 8, 128), jnp.float32),
        pltpu.SemaphoreType.DMA,
        pltpu.SemaphoreType.DMA,
        pltpu.SemaphoreType.DMA,
        pltpu.SemaphoreType.DMA,
        pltpu.SemaphoreType.DMA,
        pltpu.SemaphoreType.DMA,
    ],
    compiler_params=pltpu.CompilerParams(
        use_tc_tiling_on_sc=False, needs_layout_passes=False
    ),
)
def _gather(
    tok, wlin, out, i0, i1, g0, g1, o0, o1, sI0, sI1, sG0, sG1, sO0, sO1
):
    wid = lax.axis_index("s") * NC + lax.axis_index("c")

    def tok_slice(h):
        return tok.at[pl.ds(pl.multiple_of(h * BATCH + wid * 128, 128), 128)]

    def out_slice(h):
        return out.at[h, :, wid]

    def extract(g, o):
        for l0 in range(0, 128, 16):
            lvec = _iota16() + l0
            for d0 in range(DIM):
                dvec = lax.bitwise_and(_iota16() + d0, DIM - 1)
                vals = plsc.load_gather(g, [lvec, dvec])
                plsc.store_scatter(
                    o,
                    [
                        lax.shift_right_logical(dvec, 3),
                        lax.bitwise_and(dvec, 7),
                        lvec,
                    ],
                    vals,
                )

    pltpu.async_copy(tok_slice(0), i0, sI0)
    pltpu.async_copy(tok_slice(1), i1, sI1)
    pltpu.make_async_copy(tok_slice(0), i0, sI0).wait()
    pltpu.async_copy(wlin.at[i0], g0, sG0)

    def stage(p, h, iv, gb, ob, sI, sG, sO):
        pltpu.make_async_copy(wlin.at[iv], gb, sG).wait()

        @pl.when(h + 2 < HIST)
        def _():
            pltpu.async_copy(tok_slice(h + 2), iv, sI)

        @pl.when(p > 0)
        def _():
            pltpu.make_async_copy(ob, out_slice(h - 2), sO).wait()

        extract(gb, ob)
        pltpu.async_copy(ob, out_slice(h), sO)

    def pair_body(p, _):
        h0 = 2 * p
        pltpu.make_async_copy(tok_slice(h0 + 1), i1, sI1).wait()
        pltpu.async_copy(wlin.at[i1], g1, sG1)
        stage(p, h0, i0, g0, o0, sI0, sG0, sO0)
        @pl.when(h0 + 2 < HIST)
        def _():
            pltpu.make_async_copy(tok_slice(h0 + 2), i0, sI0).wait()
            pltpu.async_copy(wlin.at[i0], g0, sG0)

        stage(p, h0 + 1, i1, g1, o1, sI1, sG1, sO1)
        return 0

    lax.fori_loop(0, HIST // 2, pair_body, 0)
    pltpu.make_async_copy(o0, out_slice(HIST - 2), sO0).wait()
    pltpu.make_async_copy(o1, out_slice(HIST - 1), sO1).wait()


def kernel(token_ids, weight):
    wt_tail = weight[NBLK * 128 :].reshape(TAIL // PACK, 128)
    wrm = _transpose_table(weight.T, wt_tail)
    wlin = wrm.reshape(VOCAB, DIM)
    tok = token_ids.T.reshape(BATCH * HIST)
    out5 = _gather(tok, wlin)
    return out5.transpose(2, 4, 0, 1, 3).reshape(BATCH, HIST, DIM)


# speedup vs baseline: 4.9833x; 1.7681x over previous
import functools

import jax
import jax.numpy as jnp
from jax import lax
from jax.experimental import pallas as pl
from jax.experimental.pallas import tpu as pltpu
from jax.experimental.pallas import tpu_sc as plsc

BATCH = 4096
HIST = 200
DIM = 32
VOCAB = 1000000
NC, NS = 2, 16
NW = NC * NS
PACK = 128 // DIM
WRM_ROWS = VOCAB // PACK
NBLK = VOCAB // 128
TAIL = VOCAB - NBLK * 128
BLK_PER_W = NBLK // NW
BLK_REM = NBLK - BLK_PER_W * NW
BB = BATCH // 128

_MESH = plsc.VectorSubcoreMesh(core_axis_name="c", subcore_axis_name="s")


def _iota16():
    return lax.iota(jnp.int32, 16)


def _transpose_block(src, dst):
    for l0 in range(0, 128, 16):
        lvec = _iota16() + l0
        rquart = lax.shift_right_logical(lvec, 2)
        lmod = lax.bitwise_and(lvec, 3) * DIM
        for half in range(2):
            dvecs = [
                lax.bitwise_and(_iota16() + half * 16 + d0, DIM - 1)
                for d0 in range(16)
            ]
            vals = [plsc.load_gather(src, [dv, lvec]) for dv in dvecs]
            for dv, v in zip(dvecs, vals):
                plsc.store_scatter(dst, [rquart, lmod + dv], v)


@functools.partial(
    pl.kernel,
    mesh=_MESH,
    out_type=jax.ShapeDtypeStruct((WRM_ROWS, 128), jnp.float32),
    scratch_types=[
        pltpu.VMEM((DIM, 128), jnp.float32),
        pltpu.VMEM((DIM, 128), jnp.float32),
        pltpu.VMEM((DIM, 128), jnp.float32),
        pltpu.VMEM((DIM, 128), jnp.float32),
        pltpu.SemaphoreType.DMA,
        pltpu.SemaphoreType.DMA,
        pltpu.SemaphoreType.DMA,
        pltpu.SemaphoreType.DMA,
    ],
    compiler_params=pltpu.CompilerParams(needs_layout_passes=False),
)
def _transpose_table(w_t, wt_tail, wrm, s0, s1, d0, d1, si0, si1, so0, so1):
    wid = lax.axis_index("s") * NC + lax.axis_index("c")

    def in_slice(c):
        return w_t.at[:, pl.ds(pl.multiple_of(c * 128, 128), 128)]

    def out_slice(c):
        return wrm.at[pl.ds(pl.multiple_of(c * DIM, DIM), DIM), :]

    def blk(t):
        return t * NW + wid

    pltpu.async_copy(in_slice(blk(0)), s0, si0)
    pltpu.async_copy(in_slice(blk(1)), s1, si1)

    def pair_body(p, _):
        t0 = 2 * p

        def stage(t, s, d, si, so):
            pltpu.make_async_copy(in_slice(blk(t)), s, si).wait()

            @pl.when(p > 0)
            def _():
                pltpu.make_async_copy(d, out_slice(blk(t - 2)), so).wait()

            _transpose_block(s, d)
            pltpu.async_copy(d, out_slice(blk(t)), so)

            @pl.when(t + 2 < BLK_PER_W)
            def _():
                pltpu.async_copy(in_slice(blk(t + 2)), s, si)

        stage(t0, s0, d0, si0, so0)
        stage(t0 + 1, s1, d1, si1, so1)
        return 0

    lax.fori_loop(0, BLK_PER_W // 2, pair_body, 0)
    pltpu.make_async_copy(d0, out_slice(blk(BLK_PER_W - 2)), so0).wait()
    pltpu.make_async_copy(d1, out_slice(blk(BLK_PER_W - 1)), so1).wait()

    @pl.when(wid < BLK_REM)
    def _():
        c = BLK_PER_W * NW + wid
        pltpu.sync_copy(in_slice(c), s0)
        _transpose_block(s0, d0)
        pltpu.sync_copy(d0, out_slice(c))

    @pl.when(wid == NW - 1)
    def _():
        nrow = TAIL // PACK
        pltpu.sync_copy(wt_tail, s1.at[pl.ds(0, nrow), :])
        pltpu.sync_copy(
            s1.at[pl.ds(0, nrow), :], wrm.at[pl.ds(WRM_ROWS - nrow, nrow), :]
        )


@functools.partial(
    pl.kernel,
    mesh=_MESH,
    out_type=jax.ShapeDtypeStruct((HIST, PACK, BB, 8, 128), jnp.float32),
    scratch_types=[
        pltpu.VMEM((128,), jnp.int32),
        pltpu.VMEM((128,), jnp.int32),
        pltpu.VMEM((128, DIM), jnp.float32),
        pltpu.VMEM((128, DIM), jnp.float32),
        pltpu.VMEM((PACK, 8, 128), jnp.float32),
        pltpu.VMEM((PACK, 8, 128), jnp.float32),
        pltpu.SemaphoreType.DMA,
        pltpu.SemaphoreType.DMA,
        pltpu.SemaphoreType.DMA,
        pltpu.SemaphoreType.DMA,
        pltpu.SemaphoreType.DMA,
        pltpu.SemaphoreType.DMA,
    ],
    compiler_params=pltpu.CompilerParams(
        use_tc_tiling_on_sc=False, needs_layout_passes=False
    ),
)
def _gather(
    tok, wlin, out, i0, i1, g0, g1, o0, o1, sI0, sI1, sG0, sG1, sO0, sO1
):
    wid = lax.axis_index("s") * NC + lax.axis_index("c")

    def tok_slice(h):
        return tok.at[pl.ds(pl.multiple_of(h * BATCH + wid * 128, 128), 128)]

    def out_slice(h):
        return out.at[h, :, wid]

    def extract(g, o):
        for l0 in range(0, 128, 16):
            lvec = _iota16() + l0
            for half in range(2):
                dvecs = [
                    lax.bitwise_and(_iota16() + half * 16 + d0, DIM - 1)
                    for d0 in range(16)
                ]
                vals = [plsc.load_gather(g, [lvec, dv]) for dv in dvecs]
                for dv, v in zip(dvecs, vals):
                    plsc.store_scatter(
                        o,
                        [
                            lax.shift_right_logical(dv, 3),
                            lax.bitwise_and(dv, 7),
                            lvec,
                        ],
                        v,
                    )

    pltpu.async_copy(tok_slice(0), i0, sI0)
    pltpu.async_copy(tok_slice(1), i1, sI1)
    pltpu.make_async_copy(tok_slice(0), i0, sI0).wait()
    pltpu.async_copy(wlin.at[i0], g0, sG0)

    def stage(p, h, iv, gb, ob, sI, sG, sO):
        pltpu.make_async_copy(wlin.at[iv], gb, sG).wait()

        @pl.when(h + 2 < HIST)
        def _():
            pltpu.async_copy(tok_slice(h + 2), iv, sI)

        @pl.when(p > 0)
        def _():
            pltpu.make_async_copy(ob, out_slice(h - 2), sO).wait()

        extract(gb, ob)
        pltpu.async_copy(ob, out_slice(h), sO)

    def pair_body(p, _):
        h0 = 2 * p
        pltpu.make_async_copy(tok_slice(h0 + 1), i1, sI1).wait()
        pltpu.async_copy(wlin.at[i1], g1, sG1)
        stage(p, h0, i0, g0, o0, sI0, sG0, sO0)
        @pl.when(h0 + 2 < HIST)
        def _():
            pltpu.make_async_copy(tok_slice(h0 + 2), i0, sI0).wait()
            pltpu.async_copy(wlin.at[i0], g0, sG0)

        stage(p, h0 + 1, i1, g1, o1, sI1, sG1, sO1)
        return 0

    lax.fori_loop(0, HIST // 2, pair_body, 0)
    pltpu.make_async_copy(o0, out_slice(HIST - 2), sO0).wait()
    pltpu.make_async_copy(o1, out_slice(HIST - 1), sO1).wait()


def kernel(token_ids, weight):
    wt_tail = weight[NBLK * 128 :].reshape(TAIL // PACK, 128)
    wrm = _transpose_table(weight.T, wt_tail)
    wlin = wrm.reshape(VOCAB, DIM)
    tok = token_ids.T.reshape(BATCH * HIST)
    out5 = _gather(tok, wlin)
    return out5.transpose(2, 4, 0, 1, 3).reshape(BATCH, HIST, DIM)
